# Initial kernel scaffold; baseline (speedup 1.0000x reference)
import jax
import jax.numpy as jnp
from jax.experimental import pallas as pl


def kernel(r, sh_l0, sh_l1, centers, neighbors, f0_even, f0_odd, f1_even, f1_odd, U_0_p, U_0_m, U_1_p, U_1_m, W0e, W0o, W1e, W1o):
    raise NotImplementedError("write your pallas kernel here")


# speedup vs baseline: 24.1890x; 24.1890x over previous
import functools

import jax
import jax.numpy as jnp
import numpy as np
from jax import lax
from jax.experimental import pallas as pl
from jax.experimental.pallas import tpu as pltpu
from jax.experimental.pallas import tpu_sc as plsc

CUTOFF = 5.0
N = 50000
E = 800000
MP_SCALING = float(1.0 / np.sqrt(32.0))

B = 128
NBLK = 391
PER_TILE = NBLK * B
E_PAD = PER_TILE * 16
ZR = 3128
ZR_LAST = N - 15 * ZR

_C2 = -1.0 / 2.0
_C4 = 1.0 / 24.0
_C6 = -1.0 / 720.0
_C8 = 1.0 / 40320.0
_C10 = -1.0 / 3628800.0
_C12 = 1.0 / 479001600.0

_mesh = plsc.VectorSubcoreMesh(core_axis_name="c", subcore_axis_name="s")


@functools.partial(
    pl.kernel,
    mesh=_mesh,
    compiler_params=pltpu.CompilerParams(
        use_tc_tiling_on_sc=False, needs_layout_passes=False),
    out_type=jax.ShapeDtypeStruct((4 * N, 32), jnp.float32),
    scratch_types=[
        pltpu.VMEM((B,), jnp.int32),
        pltpu.VMEM((B,), jnp.int32),
        pltpu.VMEM((B,), jnp.float32),
        pltpu.VMEM((B,), jnp.float32),
        pltpu.VMEM((B,), jnp.float32),
        pltpu.VMEM((B,), jnp.float32),
        pltpu.VMEM((B,), jnp.float32),
        pltpu.VMEM((B, 32), jnp.float32),
        pltpu.VMEM((B, 32), jnp.float32),
        pltpu.VMEM((128,), jnp.float32),
        pltpu.VMEM_SHARED((N, 32), jnp.float32),
        pltpu.SemaphoreType.DMA,
    ],
)
def _sc_pool(params, cen_hbm, nbr_hbm, table, coefs_hbm, zeros_hbm, out,
             idx_v, cen_v, rv, s0v, s1av, s1bv, s1cv, g_v, c_v, coef_v,
             accum, sem):
    core = lax.axis_index("c")
    sub = lax.axis_index("s")
    iota = lax.iota(jnp.int32, 16)
    for p in range(2):
        chunk = core * 2 + p
        row0 = sub * ZR

        @pl.when(sub < 15)
        def _():
            pltpu.sync_copy(zeros_hbm, accum.at[pl.ds(row0, ZR)])

        @pl.when(sub == 15)
        def _():
            pltpu.sync_copy(zeros_hbm.at[pl.ds(0, ZR_LAST)],
                            accum.at[pl.ds(row0, ZR_LAST)])

        pltpu.sync_copy(coefs_hbm.at[pl.ds(chunk * 128, 128)], coef_v)
        plsc.subcore_barrier()
        Av = coef_v[pl.ds(0, 16)]
        Bv = coef_v[pl.ds(16, 16)]
        Cp0 = coef_v[pl.ds(32, 16)]
        Cp1 = coef_v[pl.ds(48, 16)]
        Cp2 = coef_v[pl.ds(64, 16)]
        Cm0 = coef_v[pl.ds(80, 16)]
        Cm1 = coef_v[pl.ds(96, 16)]
        Cm2 = coef_v[pl.ds(112, 16)]
        row_off = chunk * N
        base0 = sub * PER_TILE

        def blk(i, carry):
            base = base0 + i * B
            pltpu.sync_copy(params.at[pl.ds(0 * E_PAD + base, B)], rv)
            pltpu.sync_copy(params.at[pl.ds(1 * E_PAD + base, B)], s0v)
            pltpu.sync_copy(params.at[pl.ds(2 * E_PAD + base, B)], s1av)
            pltpu.sync_copy(params.at[pl.ds(3 * E_PAD + base, B)], s1bv)
            pltpu.sync_copy(params.at[pl.ds(4 * E_PAD + base, B)], s1cv)
            pltpu.sync_copy(cen_hbm.at[pl.ds(base, B)], cen_v)
            pltpu.sync_copy(nbr_hbm.at[pl.ds(base, B)], idx_v)
            for j in range(B // 16):
                sl = pl.ds(16 * j, 16)
                idx_v[sl] = idx_v[sl] + row_off
            pltpu.async_copy(table.at[idx_v], g_v, sem).wait()
            for j in range(B // 16):
                sl = pl.ds(16 * j, 16)
                rj = rv[sl]
                h0 = s0v[sl]
                h1 = s1av[sl]
                h2 = s1bv[sl]
                h3 = s1cv[sl]
                x = jnp.maximum(jnp.minimum(rj * (1.0 / CUTOFF), 1.0), 0.0)
                hh = x * (np.pi / 2.0)
                z = hh * hh
                ch = 1.0 + z * (_C2 + z * (_C4 + z * (_C6 + z * (
                    _C8 + z * (_C10 + z * _C12)))))
                fcut = ch * ch
                c1 = 2.0 * fcut - 1.0
                two = c1 + c1
                cosn = [c1]
                pm2 = jnp.full((16,), 1.0, jnp.float32)
                pm1 = c1
                for _ in range(7):
                    cur = two * pm1 - pm2
                    cosn.append(cur)
                    pm2, pm1 = pm1, cur
                bv = [cosn[t] * fcut for t in range(8)]
                q = [h0 * bv[t] for t in range(8)]
                s_e = Cp0 * h1 + Cp1 * h2 + Cp2 * h3
                t_e = Cm0 * h1 + Cm1 * h2 + Cm2 * h3
                Pe = [Av * q[t] for t in range(8)]
                Po = [Bv * q[t] for t in range(8)]
                Se = [s_e * bv[t] for t in range(4)]
                So = [t_e * bv[t] for t in range(4)]
                rows = iota + (16 * j)
                for k in range(16):
                    cke = jnp.full((16,), k, jnp.int32)
                    cko = jnp.full((16,), k + 16, jnp.int32)
                    ge = plsc.load_gather(g_v, [rows, cke])
                    go = plsc.load_gather(g_v, [rows, cko])
                    ue = Pe[k % 8] + Se[k % 4]
                    uo = Po[k % 8] + So[k % 4]
                    plsc.store_scatter(c_v, [rows, cke], ue * ge + uo * go)
                    plsc.store_scatter(c_v, [rows, cko], ue * go + uo * ge)
            pltpu.sync_copy(c_v, accum.at[cen_v], add=True)
            return carry

        lax.fori_loop(0, NBLK, blk, 0)
        plsc.subcore_barrier()

        @pl.when(sub < 15)
        def _():
            pltpu.sync_copy(accum.at[pl.ds(row0, ZR)],
                            out.at[pl.ds(row_off + row0, ZR)])

        @pl.when(sub == 15)
        def _():
            pltpu.sync_copy(accum.at[pl.ds(row0, ZR_LAST)],
                            out.at[pl.ds(row_off + row0, ZR_LAST)])

        plsc.subcore_barrier()


def _tc_body(p0, p1, p2, p3, f0e, f0o, f1e, f1o, w0e, w0o, w1e, w1o,
             o0e, o0o, o1e, o1o):
    hp = jax.lax.Precision.HIGHEST
    w0e_s = w0e[...] * MP_SCALING
    w0o_s = w0o[...] * MP_SCALING
    w1e_s = w1e[...] * MP_SCALING
    w1o_s = w1o[...] * MP_SCALING
    o0e[...] = f0e[...] + jnp.dot(p0[:, 0:16], w0e_s, precision=hp)
    o0o[...] = f0o[...] + jnp.dot(p0[:, 16:32], w0o_s, precision=hp)
    o1e[...] = f1e[...] + jnp.concatenate(
        [jnp.dot(p1[:, 0:16], w1e_s, precision=hp),
         jnp.dot(p2[:, 0:16], w1e_s, precision=hp),
         jnp.dot(p3[:, 0:16], w1e_s, precision=hp)], axis=1)
    o1o[...] = f1o[...] + jnp.concatenate(
        [jnp.dot(p1[:, 16:32], w1o_s, precision=hp),
         jnp.dot(p2[:, 16:32], w1o_s, precision=hp),
         jnp.dot(p3[:, 16:32], w1o_s, precision=hp)], axis=1)


def _tc_linear(P, f0e, f0o, f1e, f1o, w0e, w0o, w1e, w1o):
    R = 400
    G = N // R
    bs_p = pl.BlockSpec((R, 32), lambda i: (i, 0))
    bs16 = pl.BlockSpec((R, 16), lambda i: (i, 0))
    bs48 = pl.BlockSpec((R, 48), lambda i: (i, 0))
    bs_w = pl.BlockSpec((16, 16), lambda i: (0, 0))
    return pl.pallas_call(
        _tc_body,
        grid=(G,),
        in_specs=[bs_p, bs_p, bs_p, bs_p, bs16, bs16, bs48, bs48,
                  bs_w, bs_w, bs_w, bs_w],
        out_specs=[bs16, bs16, bs48, bs48],
        out_shape=[
            jax.ShapeDtypeStruct((N, 16), jnp.float32),
            jax.ShapeDtypeStruct((N, 16), jnp.float32),
            jax.ShapeDtypeStruct((N, 48), jnp.float32),
            jax.ShapeDtypeStruct((N, 48), jnp.float32),
        ],
    )(P[0:N], P[N:2 * N], P[2 * N:3 * N], P[3 * N:4 * N],
      f0e, f0o, f1e, f1o, w0e, w0o, w1e, w1o)


def kernel(r, sh_l0, sh_l1, centers, neighbors,
           f0_even, f0_odd, f1_even, f1_odd,
           U_0_p, U_0_m, U_1_p, U_1_m,
           W0e, W0o, W1e, W1o):
    pad = E_PAD - E
    params = jnp.stack([r[:, 0], sh_l0[:, 0, 0],
                        sh_l1[:, 0, 0], sh_l1[:, 1, 0], sh_l1[:, 2, 0]])
    params = jnp.pad(params, ((0, 0), (0, pad))).reshape(-1)
    cen = jnp.pad(centers.astype(jnp.int32), (0, pad))
    nbr = jnp.pad(neighbors.astype(jnp.int32), (0, pad))
    table = jnp.concatenate([
        jnp.concatenate([f0_even[:, 0, :], f0_odd[:, 0, :]], axis=1),
        jnp.concatenate([f1_even[:, 0, :], f1_odd[:, 0, :]], axis=1),
        jnp.concatenate([f1_even[:, 1, :], f1_odd[:, 1, :]], axis=1),
        jnp.concatenate([f1_even[:, 2, :], f1_odd[:, 2, :]], axis=1),
    ], axis=0)
    a = jnp.concatenate([U_0_p.reshape(1), U_1_p[:, 0]])
    b = jnp.concatenate([U_0_m.reshape(1), U_1_m[:, 0]])
    cp = jnp.concatenate([jnp.zeros((1, 3), jnp.float32), U_1_p[:, 1:4]],
                         axis=0)
    cm = jnp.concatenate([jnp.zeros((1, 3), jnp.float32), U_1_m[:, 1:4]],
                         axis=0)
    coefs = jnp.stack([a, b, cp[:, 0], cp[:, 1], cp[:, 2],
                       cm[:, 0], cm[:, 1], cm[:, 2]], axis=1)
    coefs = jnp.repeat(coefs, 16, axis=1).reshape(-1)
    zeros = jnp.zeros((ZR, 32), jnp.float32)

    P = _sc_pool(params, cen, nbr, table, coefs, zeros)

    o0e, o0o, o1e, o1o = _tc_linear(
        P, f0_even[:, 0, :], f0_odd[:, 0, :],
        f1_even.reshape(N, 48), f1_odd.reshape(N, 48),
        W0e, W0o, W1e, W1o)
    return (o0e.reshape(N, 1, 16), o0o.reshape(N, 1, 16),
            o1e.reshape(N, 3, 16), o1o.reshape(N, 3, 16))


# speedup vs baseline: 37.2634x; 1.5405x over previous
import functools

import jax
import jax.numpy as jnp
import numpy as np
from jax import lax
from jax.experimental import pallas as pl
from jax.experimental.pallas import tpu as pltpu
from jax.experimental.pallas import tpu_sc as plsc

CUTOFF = 5.0
N = 50000
E = 800000
MP_SCALING = float(1.0 / np.sqrt(32.0))

B = 128
NBLK = 392
PER_TILE = NBLK * B
E_PAD = PER_TILE * 16
ZR = 3128
ZR_LAST = N - 15 * ZR

_C2 = -1.0 / 2.0
_C4 = 1.0 / 24.0
_C6 = -1.0 / 720.0
_C8 = 1.0 / 40320.0
_C10 = -1.0 / 3628800.0
_C12 = 1.0 / 479001600.0

_mesh = plsc.VectorSubcoreMesh(core_axis_name="c", subcore_axis_name="s")


@functools.partial(
    pl.kernel,
    mesh=_mesh,
    compiler_params=pltpu.CompilerParams(
        use_tc_tiling_on_sc=False, needs_layout_passes=False),
    out_type=jax.ShapeDtypeStruct((4 * N, 32), jnp.float32),
    scratch_types=(
        [pltpu.VMEM((B * 8,), jnp.int32)] * 4 +
        [pltpu.VMEM((B,), jnp.int32)] * 4 +
        [pltpu.VMEM((B,), jnp.int32)] * 2 +
        [pltpu.VMEM((B, 32), jnp.float32)] * 4 +
        [pltpu.VMEM((B, 32), jnp.float32)] * 2 +
        [pltpu.VMEM((128,), jnp.float32)] +
        [pltpu.VMEM_SHARED((N, 32), jnp.float32)] +
        [pltpu.SemaphoreType.DMA] * 10
    ),
)
def _sc_pool(edges, table, coefs_hbm, zeros_hbm, out,
             eb0, eb1, eb2, eb3, ix0, ix1, ix2, ix3, cs0, cs1,
             g0, g1, g2, g3, cc0, cc1, coef_v, accum,
             se0, se1, se2, se3, sg0, sg1, sg2, sg3, ss0, ss1):
    ebuf = [eb0, eb1, eb2, eb3]
    idx = [ix0, ix1, ix2, ix3]
    cen_s = [cs0, cs1]
    gb = [g0, g1, g2, g3]
    cb = [cc0, cc1]
    sem_e = [se0, se1, se2, se3]
    sem_g = [sg0, sg1, sg2, sg3]
    sem_s = [ss0, ss1]
    core = lax.axis_index("c")
    sub = lax.axis_index("s")
    iota = lax.iota(jnp.int32, 16)
    iota8 = iota * 8
    for p in range(2):
        chunk = core * 2 + p
        row0 = sub * ZR

        @pl.when(sub < 15)
        def _():
            pltpu.sync_copy(zeros_hbm, accum.at[pl.ds(row0, ZR)])

        @pl.when(sub == 15)
        def _():
            pltpu.sync_copy(zeros_hbm.at[pl.ds(0, ZR_LAST)],
                            accum.at[pl.ds(row0, ZR_LAST)])

        pltpu.sync_copy(coefs_hbm.at[pl.ds(chunk * 128, 128)], coef_v)
        plsc.subcore_barrier()
        row_off = chunk * N
        base0 = sub * PER_TILE

        def start_e(k, b):
            pltpu.async_copy(
                edges.at[pl.ds((base0 + k * B) * 8, B * 8)],
                ebuf[b], sem_e[b])

        def u_phase(k, b):
            pltpu.make_async_copy(
                edges.at[pl.ds(0, B * 8)], ebuf[b], sem_e[b]).wait()
            for j in range(B // 16):
                nb = plsc.load_gather(ebuf[b], [iota8 + (128 * j + 6)])
                idx[b][pl.ds(16 * j, 16)] = nb + row_off
            pltpu.async_copy(table.at[idx[b]], gb[b], sem_g[b])

        def c_phase(b, cs, do_swait):
            pltpu.make_async_copy(
                table.at[idx[b]], gb[b], sem_g[b]).wait()
            if do_swait:
                pltpu.make_async_copy(
                    cb[cs], accum.at[cen_s[cs]], sem_s[cs]).wait()
            Av = coef_v[pl.ds(0, 16)]
            Bv = coef_v[pl.ds(16, 16)]
            Cp0 = coef_v[pl.ds(32, 16)]
            Cp1 = coef_v[pl.ds(48, 16)]
            Cp2 = coef_v[pl.ds(64, 16)]
            Cm0 = coef_v[pl.ds(80, 16)]
            Cm1 = coef_v[pl.ds(96, 16)]
            Cm2 = coef_v[pl.ds(112, 16)]

            def grp(j, carry):
                eb = iota8 + 128 * j
                rj = plsc.bitcast(
                    plsc.load_gather(ebuf[b], [eb]), jnp.float32)
                h0 = plsc.bitcast(
                    plsc.load_gather(ebuf[b], [eb + 1]), jnp.float32)
                h1 = plsc.bitcast(
                    plsc.load_gather(ebuf[b], [eb + 2]), jnp.float32)
                h2 = plsc.bitcast(
                    plsc.load_gather(ebuf[b], [eb + 3]), jnp.float32)
                h3 = plsc.bitcast(
                    plsc.load_gather(ebuf[b], [eb + 4]), jnp.float32)
                cen_s[cs][pl.ds(16 * j, 16)] = plsc.load_gather(
                    ebuf[b], [eb + 5])
                x = jnp.maximum(jnp.minimum(rj * (1.0 / CUTOFF), 1.0), 0.0)
                hh = x * (np.pi / 2.0)
                z = hh * hh
                ch = 1.0 + z * (_C2 + z * (_C4 + z * (_C6 + z * (
                    _C8 + z * (_C10 + z * _C12)))))
                fcut = ch * ch
                c1 = 2.0 * fcut - 1.0
                two = c1 + c1
                cosn = [c1]
                pm2 = jnp.full((16,), 1.0, jnp.float32)
                pm1 = c1
                for _ in range(7):
                    cur = two * pm1 - pm2
                    cosn.append(cur)
                    pm2, pm1 = pm1, cur
                bv = [cosn[t] * fcut for t in range(8)]
                q = [h0 * bv[t] for t in range(8)]
                s_e = Cp0 * h1 + Cp1 * h2 + Cp2 * h3
                t_e = Cm0 * h1 + Cm1 * h2 + Cm2 * h3
                Se = [s_e * bv[t] for t in range(4)]
                So = [t_e * bv[t] for t in range(4)]
                rows = iota + 16 * j
                cke = rows - rows
                for k in range(16):
                    cko = cke + 16
                    ge = plsc.load_gather(gb[b], [rows, cke])
                    go = plsc.load_gather(gb[b], [rows, cko])
                    ue = Av * q[k % 8] + Se[k % 4]
                    uo = Bv * q[k % 8] + So[k % 4]
                    plsc.store_scatter(cb[cs], [rows, cke], ue * ge + uo * go)
                    plsc.store_scatter(cb[cs], [rows, cko], ue * go + uo * ge)
                    cke = cke + 1
                return carry

            lax.fori_loop(0, B // 16, grp, 0)
            pltpu.async_copy(cb[cs], accum.at[cen_s[cs]], sem_s[cs],
                             add=True)

        def quad(i, first):
            for b in range(4):
                k = 4 * i + b
                c_phase(b, b % 2, not (first and b < 2))
                k2 = k + 2
                b2 = (b + 2) % 4

                @pl.when(k2 < NBLK)
                def _():
                    u_phase(k2, b2)

                @pl.when(k + 4 < NBLK)
                def _():
                    start_e(k + 4, b)

        for kk in range(4):
            start_e(kk, kk)
        u_phase(0, 0)
        u_phase(1, 1)
        quad(jnp.int32(0), True)
        lax.fori_loop(1, NBLK // 4,
                      lambda i, c: (quad(i, False), c)[1], 0)
        pltpu.make_async_copy(cb[0], accum.at[cen_s[0]], sem_s[0]).wait()
        pltpu.make_async_copy(cb[1], accum.at[cen_s[1]], sem_s[1]).wait()
        plsc.subcore_barrier()

        @pl.when(sub < 15)
        def _():
            pltpu.sync_copy(accum.at[pl.ds(row0, ZR)],
                            out.at[pl.ds(row_off + row0, ZR)])

        @pl.when(sub == 15)
        def _():
            pltpu.sync_copy(accum.at[pl.ds(row0, ZR_LAST)],
                            out.at[pl.ds(row_off + row0, ZR_LAST)])

        plsc.subcore_barrier()


def _tc_body(p0, p1, p2, p3, f0e, f0o, f1e, f1o, w0e, w0o, w1e, w1o,
             o0e, o0o, o1e, o1o):
    hp = jax.lax.Precision.HIGHEST
    w0e_s = w0e[...] * MP_SCALING
    w0o_s = w0o[...] * MP_SCALING
    w1e_s = w1e[...] * MP_SCALING
    w1o_s = w1o[...] * MP_SCALING
    o0e[...] = f0e[...] + jnp.dot(p0[:, 0:16], w0e_s, precision=hp)
    o0o[...] = f0o[...] + jnp.dot(p0[:, 16:32], w0o_s, precision=hp)
    o1e[...] = f1e[...] + jnp.concatenate(
        [jnp.dot(p1[:, 0:16], w1e_s, precision=hp),
         jnp.dot(p2[:, 0:16], w1e_s, precision=hp),
         jnp.dot(p3[:, 0:16], w1e_s, precision=hp)], axis=1)
    o1o[...] = f1o[...] + jnp.concatenate(
        [jnp.dot(p1[:, 16:32], w1o_s, precision=hp),
         jnp.dot(p2[:, 16:32], w1o_s, precision=hp),
         jnp.dot(p3[:, 16:32], w1o_s, precision=hp)], axis=1)


def _tc_linear(P, f0e, f0o, f1e, f1o, w0e, w0o, w1e, w1o):
    R = 400
    G = N // R
    bs_p = pl.BlockSpec((R, 32), lambda i: (i, 0))
    bs16 = pl.BlockSpec((R, 16), lambda i: (i, 0))
    bs48 = pl.BlockSpec((R, 48), lambda i: (i, 0))
    bs_w = pl.BlockSpec((16, 16), lambda i: (0, 0))
    return pl.pallas_call(
        _tc_body,
        grid=(G,),
        in_specs=[bs_p, bs_p, bs_p, bs_p, bs16, bs16, bs48, bs48,
                  bs_w, bs_w, bs_w, bs_w],
        out_specs=[bs16, bs16, bs48, bs48],
        out_shape=[
            jax.ShapeDtypeStruct((N, 16), jnp.float32),
            jax.ShapeDtypeStruct((N, 16), jnp.float32),
            jax.ShapeDtypeStruct((N, 48), jnp.float32),
            jax.ShapeDtypeStruct((N, 48), jnp.float32),
        ],
    )(P[0:N], P[N:2 * N], P[2 * N:3 * N], P[3 * N:4 * N],
      f0e, f0o, f1e, f1o, w0e, w0o, w1e, w1o)


def kernel(r, sh_l0, sh_l1, centers, neighbors,
           f0_even, f0_odd, f1_even, f1_odd,
           U_0_p, U_0_m, U_1_p, U_1_m,
           W0e, W0o, W1e, W1o):
    pad = E_PAD - E
    fpart = jnp.stack([r[:, 0], sh_l0[:, 0, 0],
                       sh_l1[:, 0, 0], sh_l1[:, 1, 0], sh_l1[:, 2, 0]],
                      axis=1)
    fbits = jax.lax.bitcast_convert_type(fpart, jnp.int32)
    cen = centers.astype(jnp.int32)
    ipart = jnp.stack([cen, neighbors.astype(jnp.int32),
                       jnp.zeros_like(cen)], axis=1)
    edges = jnp.concatenate([fbits, ipart], axis=1)
    edges = jnp.pad(edges, ((0, pad), (0, 0))).reshape(-1)
    table = jnp.concatenate([
        jnp.concatenate([f0_even[:, 0, :], f0_odd[:, 0, :]], axis=1),
        jnp.concatenate([f1_even[:, 0, :], f1_odd[:, 0, :]], axis=1),
        jnp.concatenate([f1_even[:, 1, :], f1_odd[:, 1, :]], axis=1),
        jnp.concatenate([f1_even[:, 2, :], f1_odd[:, 2, :]], axis=1),
    ], axis=0)
    a = jnp.concatenate([U_0_p.reshape(1), U_1_p[:, 0]])
    b = jnp.concatenate([U_0_m.reshape(1), U_1_m[:, 0]])
    cp = jnp.concatenate([jnp.zeros((1, 3), jnp.float32), U_1_p[:, 1:4]],
                         axis=0)
    cm = jnp.concatenate([jnp.zeros((1, 3), jnp.float32), U_1_m[:, 1:4]],
                         axis=0)
    coefs = jnp.stack([a, b, cp[:, 0], cp[:, 1], cp[:, 2],
                       cm[:, 0], cm[:, 1], cm[:, 2]], axis=1)
    coefs = jnp.repeat(coefs, 16, axis=1).reshape(-1)
    zeros = jnp.zeros((ZR, 32), jnp.float32)

    P = _sc_pool(edges, table, coefs, zeros)

    o0e, o0o, o1e, o1o = _tc_linear(
        P, f0_even[:, 0, :], f0_odd[:, 0, :],
        f1_even.reshape(N, 48), f1_odd.reshape(N, 48),
        W0e, W0o, W1e, W1o)
    return (o0e.reshape(N, 1, 16), o0o.reshape(N, 1, 16),
            o1e.reshape(N, 3, 16), o1o.reshape(N, 3, 16))
